# baseline (speedup 1.0000x reference)
import functools

import jax
import jax.numpy as jnp
from jax import lax
from jax.experimental import pallas as pl
from jax.experimental.pallas import tpu as pltpu
from jax.experimental.pallas import tpu_sc as plsc

N = 10000
D = 128
H = 128
E = 320000

NC = 2
NS = 16
NW = NC * NS
CHUNK = 128
NCH = 80
EPW = NCH * CHUNK
EPAD = NW * EPW
NROW = 10112
RPT = NROW // NS

_LP = (0.9999918285309969, -0.4993725978465231, 0.32529514140155963,
       -0.21029369270421422, 0.10150004715404037, -0.02397957307223611)


def _softplus_sc(x):
    t = jnp.exp(-jnp.abs(x))
    p = jnp.float32(_LP[5])
    for c in (_LP[4], _LP[3], _LP[2], _LP[1], _LP[0]):
        p = p * t + jnp.float32(c)
    return jnp.maximum(x, 0.0) + t * p


def _d2_body(px, py, pz, row_hbm, col_hbm, d2_hbm,
             pxv, pyv, pzv, rowv, colv, outv):
    c = lax.axis_index("c")
    s = lax.axis_index("s")
    wid = s * NC + c
    pltpu.sync_copy(px, pxv)
    pltpu.sync_copy(py, pyv)
    pltpu.sync_copy(pz, pzv)
    pltpu.sync_copy(row_hbm.at[wid], rowv)
    pltpu.sync_copy(col_hbm.at[wid], colv)

    @pl.loop(0, EPW // 16)
    def _(t):
        j = t // 8
        k = (t % 8) * 16
        r = rowv[j, 0, pl.ds(k, 16)]
        cc = colv[j, 0, pl.ds(k, 16)]
        dx = plsc.load_gather(pxv, [r]) - plsc.load_gather(pxv, [cc])
        dy = plsc.load_gather(pyv, [r]) - plsc.load_gather(pyv, [cc])
        dz = plsc.load_gather(pzv, [r]) - plsc.load_gather(pzv, [cc])
        outv[j, pl.ds(k, 16)] = dx * dx + dy * dy + dz * dz

    pltpu.sync_copy(outv, d2_hbm.at[wid])


_d2_call = functools.partial(
    pl.kernel,
    out_type=jax.ShapeDtypeStruct((NW, NCH, CHUNK), jnp.float32),
    compiler_params=pltpu.CompilerParams(needs_layout_passes=False),
    mesh=plsc.VectorSubcoreMesh(core_axis_name="c", subcore_axis_name="s"),
    scratch_types=[
        pltpu.VMEM((NROW,), jnp.float32),
        pltpu.VMEM((NROW,), jnp.float32),
        pltpu.VMEM((NROW,), jnp.float32),
        pltpu.VMEM((NCH, 1, CHUNK), jnp.int32),
        pltpu.VMEM((NCH, 1, CHUNK), jnp.int32),
        pltpu.VMEM((NCH, CHUNK), jnp.float32),
    ],
)(_d2_body)


def _conv_body(h_hbm, row_hbm, col_hbm, ea_hbm, zeros_hbm, out_hbm,
               rowv, colv, gbuf, eabuf, aggsh, gsem):
    c = lax.axis_index("c")
    s = lax.axis_index("s")
    wid = s * NC + c
    pltpu.sync_copy(zeros_hbm.at[pl.ds(s * RPT, RPT)],
                    aggsh.at[pl.ds(s * RPT, RPT)])
    plsc.subcore_barrier()

    @pl.loop(0, NCH)
    def _(j):
        pltpu.sync_copy(row_hbm.at[wid, j], rowv)
        pltpu.sync_copy(col_hbm.at[wid, j], colv)
        pltpu.async_copy(h_hbm.at[rowv.at[0]], gbuf, gsem).wait()
        pltpu.sync_copy(ea_hbm.at[wid, j], eabuf)

        @pl.loop(0, CHUNK * H // 16, unroll=4)
        def _(t):
            i = t // (H // 16)
            k = (t % (H // 16)) * 16
            x = gbuf[i, pl.ds(k, 16)] + eabuf[i, pl.ds(k, 16)]
            eabuf[i, pl.ds(k, 16)] = _softplus_sc(x)

        pltpu.sync_copy(eabuf, aggsh.at[colv.at[0]], add=True)

    plsc.subcore_barrier()
    pltpu.sync_copy(aggsh.at[pl.ds(s * RPT, RPT)],
                    out_hbm.at[c, pl.ds(s * RPT, RPT)])


_conv_call = functools.partial(
    pl.kernel,
    out_type=jax.ShapeDtypeStruct((NC, NROW, H), jnp.float32),
    compiler_params=pltpu.CompilerParams(needs_layout_passes=False),
    mesh=plsc.VectorSubcoreMesh(core_axis_name="c", subcore_axis_name="s"),
    scratch_types=[
        pltpu.VMEM((1, CHUNK), jnp.int32),
        pltpu.VMEM((1, CHUNK), jnp.int32),
        pltpu.VMEM((CHUNK, H), jnp.float32),
        pltpu.VMEM((CHUNK, H), jnp.float32),
        pltpu.VMEM_SHARED((NROW, H), jnp.float32),
        pltpu.SemaphoreType.DMA,
    ],
)(_conv_body)


def _embed_body(x_ref, w_ref, b_ref, o_ref):
    o_ref[...] = (jnp.dot(x_ref[...], w_ref[...],
                          preferred_element_type=jnp.float32) + b_ref[...])


def _embed(x, w, b):
    return pl.pallas_call(
        _embed_body,
        grid=(5,),
        in_specs=[
            pl.BlockSpec((2000, D), lambda i: (i, 0)),
            pl.BlockSpec((D, H), lambda i: (0, 0)),
            pl.BlockSpec((1, H), lambda i: (0, 0)),
        ],
        out_specs=pl.BlockSpec((2000, H), lambda i: (i, 0)),
        out_shape=jax.ShapeDtypeStruct((N, H), jnp.float32),
    )(x, w, b)


def _ea_body(d2_ref, w0_ref, b0_ref, w1_ref, b1_ref, o_ref):
    w0 = w0_ref[...]
    b0 = b0_ref[...]
    w1 = w1_ref[...]
    b1 = b1_ref[...]
    for r in range(8):
        drow = d2_ref[r:r + 1, :]
        dcol = jnp.broadcast_to(drow, (128, 128)).T
        u = jnp.maximum(dcol * w0 + b0, 0.0)
        o_ref[pl.ds(r * 128, 128), :] = (
            jnp.dot(u, w1, preferred_element_type=jnp.float32) + b1)


def _ea(d2r, w0, b0, w1, b1):
    return pl.pallas_call(
        _ea_body,
        grid=(EPAD // 1024,),
        in_specs=[
            pl.BlockSpec((8, 128), lambda i: (i, 0)),
            pl.BlockSpec((1, H), lambda i: (0, 0)),
            pl.BlockSpec((1, H), lambda i: (0, 0)),
            pl.BlockSpec((H, H), lambda i: (0, 0)),
            pl.BlockSpec((1, H), lambda i: (0, 0)),
        ],
        out_specs=pl.BlockSpec((1024, H), lambda i: (i, 0)),
        out_shape=jax.ShapeDtypeStruct((EPAD, H), jnp.float32),
    )(d2r, w0, b0, w1, b1)


def _mlp_body(agg_ref, h_ref, w0_ref, b0_ref, w1_ref, b1_ref, o_ref):
    a = agg_ref[0] + agg_ref[1] + h_ref[...]
    z = jax.nn.softplus(jnp.dot(a, w0_ref[...],
                                preferred_element_type=jnp.float32) + b0_ref[...])
    o_ref[...] = jax.nn.softplus(
        jnp.dot(z, w1_ref[...], preferred_element_type=jnp.float32) + b1_ref[...])


def _mlp(agg, h, w0, b0, w1, b1):
    return pl.pallas_call(
        _mlp_body,
        grid=(5,),
        in_specs=[
            pl.BlockSpec((NC, 2000, H), lambda i: (0, i, 0)),
            pl.BlockSpec((2000, H), lambda i: (i, 0)),
            pl.BlockSpec((H, H), lambda i: (0, 0)),
            pl.BlockSpec((1, H), lambda i: (0, 0)),
            pl.BlockSpec((H, H), lambda i: (0, 0)),
            pl.BlockSpec((1, H), lambda i: (0, 0)),
        ],
        out_specs=pl.BlockSpec((2000, H), lambda i: (i, 0)),
        out_shape=jax.ShapeDtypeStruct((N, H), jnp.float32),
    )(agg, h, w0, b0, w1, b1)


def _mlp_pool_body(agg_ref, h_ref, w0_ref, b0_ref, w1_ref, b1_ref, m_ref,
                   part_ref, msum_ref):
    a = agg_ref[0] + agg_ref[1] + h_ref[...]
    z = jax.nn.softplus(jnp.dot(a, w0_ref[...],
                                preferred_element_type=jnp.float32) + b0_ref[...])
    z = jnp.dot(z, w1_ref[...], preferred_element_type=jnp.float32) + b1_ref[...]
    m = m_ref[...]
    part_ref[0] = jnp.sum(z * m, axis=0, keepdims=True)
    msum_ref[0] = jnp.sum(m, axis=0, keepdims=True)


def _mlp_pool(agg, h, w0, b0, w1, b1, mask_b):
    return pl.pallas_call(
        _mlp_pool_body,
        grid=(5,),
        in_specs=[
            pl.BlockSpec((NC, 2000, H), lambda i: (0, i, 0)),
            pl.BlockSpec((2000, H), lambda i: (i, 0)),
            pl.BlockSpec((H, H), lambda i: (0, 0)),
            pl.BlockSpec((1, H), lambda i: (0, 0)),
            pl.BlockSpec((H, H), lambda i: (0, 0)),
            pl.BlockSpec((1, H), lambda i: (0, 0)),
            pl.BlockSpec((2000, H), lambda i: (i, 0)),
        ],
        out_specs=[
            pl.BlockSpec((1, 1, H), lambda i: (i, 0, 0)),
            pl.BlockSpec((1, 1, H), lambda i: (i, 0, 0)),
        ],
        out_shape=[
            jax.ShapeDtypeStruct((5, 1, H), jnp.float32),
            jax.ShapeDtypeStruct((5, 1, H), jnp.float32),
        ],
    )(agg, h, w0, b0, w1, b1, mask_b)


def _proj_body(part_ref, msum_ref, w0_ref, b0_ref, w1_ref, b1_ref, o_ref):
    pooled = jnp.sum(part_ref[...], axis=0) / jnp.sum(msum_ref[...], axis=0)
    pr = jnp.broadcast_to(pooled, (8, H))
    y = jnp.maximum(jnp.dot(pr, w0_ref[...],
                            preferred_element_type=jnp.float32) + b0_ref[...], 0.0)
    o_ref[...] = jnp.dot(y, w1_ref[...],
                         preferred_element_type=jnp.float32) + b1_ref[...]


def _proj(part, msum, w0, b0, w1p, b1p):
    return pl.pallas_call(
        _proj_body,
        out_shape=jax.ShapeDtypeStruct((8, H), jnp.float32),
    )(part, msum, w0, b0, w1p, b1p)


def kernel(node_features, batch_mask, pos, edge_index, node_W, node_b,
           edge_W0, edge_b0, edge_W1, edge_b1,
           c0_W0, c0_b0, c0_W1, c0_b1, c1_W0, c1_b0, c1_W1, c1_b1,
           c2_W0, c2_b0, c2_W1, c2_b1, proj_W0, proj_b0, proj_W1, proj_b1):
    x = node_features.reshape(N, D)
    p = pos.reshape(N, 3)
    px = jnp.pad(p[:, 0], (0, NROW - N))
    py = jnp.pad(p[:, 1], (0, NROW - N))
    pz = jnp.pad(p[:, 2], (0, NROW - N))

    row = edge_index[0]
    col = edge_index[1]
    pad = EPAD - E
    rowp = jnp.concatenate([row, jnp.zeros((pad,), jnp.int32)]).reshape(NW, NCH, 1, CHUNK)
    colp = jnp.concatenate([col, jnp.full((pad,), N, jnp.int32)]).reshape(NW, NCH, 1, CHUNK)

    h = _embed(x, node_W, node_b.reshape(1, H))
    d2 = _d2_call(px, py, pz, rowp, colp)
    ea = _ea(d2.reshape(EPAD // 128, 128), edge_W0.reshape(1, H),
             edge_b0.reshape(1, H), edge_W1, edge_b1.reshape(1, H))
    ea4 = ea.reshape(NW, NCH, CHUNK, H)

    zeros = jnp.zeros((NROW, H), jnp.float32)
    mask_b = jnp.broadcast_to(batch_mask.reshape(N, 1), (N, H))

    convs = [(c0_W0, c0_b0, c0_W1, c0_b1),
             (c1_W0, c1_b0, c1_W1, c1_b1),
             (c2_W0, c2_b0, c2_W1, c2_b1)]

    for i, (w0, b0, w1, b1) in enumerate(convs):
        agg = _conv_call(h, rowp, colp, ea4, zeros)
        if i < 2:
            h = _mlp(agg, h, w0, b0.reshape(1, H), w1, b1.reshape(1, H))
        else:
            part, msum = _mlp_pool(agg, h, w0, b0.reshape(1, H),
                                   w1, b1.reshape(1, H), mask_b)

    w1p = jnp.pad(proj_W1, ((0, 0), (0, H - 1)))
    b1p = jnp.pad(proj_b1.reshape(1, 1), ((0, 0), (0, H - 1)))
    out = _proj(part, msum, proj_W0, proj_b0.reshape(1, H), w1p, b1p)
    return out[0:1, 0:1]


# speedup vs baseline: 1.2545x; 1.2545x over previous
import functools

import jax
import jax.numpy as jnp
from jax import lax
from jax.experimental import pallas as pl
from jax.experimental.pallas import tpu as pltpu
from jax.experimental.pallas import tpu_sc as plsc

N = 10000
D = 128
H = 128
E = 320000

NC = 2
NS = 16
NW = NC * NS
CHUNK = 64
NCH = 160
EPW = NCH * CHUNK
EPAD = NW * EPW
NROW = 10112
RPT = NROW // NS

_LP = (0.9999918285309969, -0.4993725978465231, 0.32529514140155963,
       -0.21029369270421422, 0.10150004715404037, -0.02397957307223611)


def _softplus_sc(x):
    t = jnp.exp(-jnp.abs(x))
    p = jnp.float32(_LP[5])
    for c in (_LP[4], _LP[3], _LP[2], _LP[1], _LP[0]):
        p = p * t + jnp.float32(c)
    return jnp.maximum(x, 0.0) + t * p


def _d2_body(px, py, pz, row_hbm, col_hbm, d2_hbm,
             pxv, pyv, pzv, rowv, colv, outv):
    c = lax.axis_index("c")
    s = lax.axis_index("s")
    wid = s * NC + c
    pltpu.sync_copy(px, pxv)
    pltpu.sync_copy(py, pyv)
    pltpu.sync_copy(pz, pzv)
    pltpu.sync_copy(row_hbm.at[wid], rowv)
    pltpu.sync_copy(col_hbm.at[wid], colv)

    @pl.loop(0, EPW // 16)
    def _(t):
        j = t // (CHUNK // 16)
        k = (t % (CHUNK // 16)) * 16
        r = rowv[j, 0, pl.ds(k, 16)]
        cc = colv[j, 0, pl.ds(k, 16)]
        dx = plsc.load_gather(pxv, [r]) - plsc.load_gather(pxv, [cc])
        dy = plsc.load_gather(pyv, [r]) - plsc.load_gather(pyv, [cc])
        dz = plsc.load_gather(pzv, [r]) - plsc.load_gather(pzv, [cc])
        outv[j, pl.ds(k, 16)] = dx * dx + dy * dy + dz * dz

    pltpu.sync_copy(outv, d2_hbm.at[wid])


_d2_call = functools.partial(
    pl.kernel,
    out_type=jax.ShapeDtypeStruct((NW, NCH, CHUNK), jnp.float32),
    compiler_params=pltpu.CompilerParams(needs_layout_passes=False),
    mesh=plsc.VectorSubcoreMesh(core_axis_name="c", subcore_axis_name="s"),
    scratch_types=[
        pltpu.VMEM((NROW,), jnp.float32),
        pltpu.VMEM((NROW,), jnp.float32),
        pltpu.VMEM((NROW,), jnp.float32),
        pltpu.VMEM((NCH, 1, CHUNK), jnp.int32),
        pltpu.VMEM((NCH, 1, CHUNK), jnp.int32),
        pltpu.VMEM((NCH, CHUNK), jnp.float32),
    ],
)(_d2_body)


def _conv_body(h_hbm, row_hbm, col_hbm, ea_hbm, zeros_hbm, out_hbm,
               rowb, colb, gbuf, eabuf, mbuf, aggsh,
               isem, gsem, easem, ssem):
    c = lax.axis_index("c")
    s = lax.axis_index("s")
    wid = s * NC + c
    pltpu.sync_copy(zeros_hbm.at[pl.ds(s * RPT, RPT)],
                    aggsh.at[pl.ds(s * RPT, RPT)])
    plsc.subcore_barrier()

    def issue_idx(j, q):
        pltpu.async_copy(row_hbm.at[wid, j], rowb.at[q], isem.at[q])
        pltpu.async_copy(col_hbm.at[wid, j], colb.at[q], isem.at[q])

    def wait_idx(j, q):
        pltpu.make_async_copy(row_hbm.at[wid, j], rowb.at[q], isem.at[q]).wait()
        pltpu.make_async_copy(col_hbm.at[wid, j], colb.at[q], isem.at[q]).wait()

    def issue_gea(j, q, b):
        pltpu.async_copy(h_hbm.at[rowb.at[q, 0]], gbuf.at[b], gsem.at[b])
        pltpu.async_copy(ea_hbm.at[wid, j], eabuf.at[b], easem.at[b])

    def wait_gea(j, b):
        pltpu.make_async_copy(ea_hbm.at[wid, j], gbuf.at[b], gsem.at[b]).wait()
        pltpu.make_async_copy(ea_hbm.at[wid, j], eabuf.at[b], easem.at[b]).wait()

    def compute(b):
        @pl.loop(0, CHUNK * H // 16, unroll=8)
        def _(t):
            i = t // (H // 16)
            k = (t % (H // 16)) * 16
            x = gbuf[b, i, pl.ds(k, 16)] + eabuf[b, i, pl.ds(k, 16)]
            mbuf[b, i, pl.ds(k, 16)] = _softplus_sc(x)

    def issue_scat(q, b):
        pltpu.async_copy(mbuf.at[b], aggsh.at[colb.at[q, 0]], ssem.at[b],
                         add=True)

    def wait_scat(b):
        pltpu.make_async_copy(ea_hbm.at[wid, 0], mbuf.at[b], ssem.at[b]).wait()

    issue_idx(0, 0)
    issue_idx(1, 1)
    wait_idx(0, 0)
    issue_gea(0, 0, 0)

    @pl.loop(0, NCH, step=4)
    def _(jj):
        for u in range(4):
            j = jj + u
            b = u & 1
            b1 = 1 - b

            @pl.when(j >= 2)
            def _():
                wait_scat(b)

            @pl.when(j + 2 < NCH)
            def _():
                issue_idx(j + 2, (u + 2) & 3)

            @pl.when(j + 1 < NCH)
            def _():
                wait_idx(j + 1, (u + 1) & 3)
                issue_gea(j + 1, (u + 1) & 3, b1)

            wait_gea(j, b)
            compute(b)
            issue_scat(u, b)

    wait_scat(0)
    wait_scat(1)
    plsc.subcore_barrier()
    pltpu.sync_copy(aggsh.at[pl.ds(s * RPT, RPT)],
                    out_hbm.at[c, pl.ds(s * RPT, RPT)])


_conv_call = functools.partial(
    pl.kernel,
    out_type=jax.ShapeDtypeStruct((NC, NROW, H), jnp.float32),
    compiler_params=pltpu.CompilerParams(needs_layout_passes=False),
    mesh=plsc.VectorSubcoreMesh(core_axis_name="c", subcore_axis_name="s"),
    scratch_types=[
        pltpu.VMEM((4, 1, CHUNK), jnp.int32),
        pltpu.VMEM((4, 1, CHUNK), jnp.int32),
        pltpu.VMEM((2, CHUNK, H), jnp.float32),
        pltpu.VMEM((2, CHUNK, H), jnp.float32),
        pltpu.VMEM((2, CHUNK, H), jnp.float32),
        pltpu.VMEM_SHARED((NROW, H), jnp.float32),
        pltpu.SemaphoreType.DMA((4,)),
        pltpu.SemaphoreType.DMA((2,)),
        pltpu.SemaphoreType.DMA((2,)),
        pltpu.SemaphoreType.DMA((2,)),
    ],
)(_conv_body)


def _embed_body(x_ref, w_ref, b_ref, o_ref):
    o_ref[...] = (jnp.dot(x_ref[...], w_ref[...],
                          preferred_element_type=jnp.float32) + b_ref[...])


def _embed(x, w, b):
    return pl.pallas_call(
        _embed_body,
        grid=(5,),
        in_specs=[
            pl.BlockSpec((2000, D), lambda i: (i, 0)),
            pl.BlockSpec((D, H), lambda i: (0, 0)),
            pl.BlockSpec((1, H), lambda i: (0, 0)),
        ],
        out_specs=pl.BlockSpec((2000, H), lambda i: (i, 0)),
        out_shape=jax.ShapeDtypeStruct((N, H), jnp.float32),
    )(x, w, b)


def _ea_body(d2_ref, w0_ref, b0_ref, w1_ref, b1_ref, o_ref):
    w0 = w0_ref[...]
    b0 = b0_ref[...]
    w1 = w1_ref[...]
    b1 = b1_ref[...]
    for r in range(8):
        drow = d2_ref[r:r + 1, :]
        dcol = jnp.broadcast_to(drow, (128, 128)).T
        u = jnp.maximum(dcol * w0 + b0, 0.0)
        o_ref[pl.ds(r * 128, 128), :] = (
            jnp.dot(u, w1, preferred_element_type=jnp.float32) + b1)


def _ea(d2r, w0, b0, w1, b1):
    return pl.pallas_call(
        _ea_body,
        grid=(EPAD // 1024,),
        in_specs=[
            pl.BlockSpec((8, 128), lambda i: (i, 0)),
            pl.BlockSpec((1, H), lambda i: (0, 0)),
            pl.BlockSpec((1, H), lambda i: (0, 0)),
            pl.BlockSpec((H, H), lambda i: (0, 0)),
            pl.BlockSpec((1, H), lambda i: (0, 0)),
        ],
        out_specs=pl.BlockSpec((1024, H), lambda i: (i, 0)),
        out_shape=jax.ShapeDtypeStruct((EPAD, H), jnp.float32),
    )(d2r, w0, b0, w1, b1)


def _mlp_body(agg_ref, h_ref, w0_ref, b0_ref, w1_ref, b1_ref, o_ref):
    a = agg_ref[0] + agg_ref[1] + h_ref[...]
    z = jax.nn.softplus(jnp.dot(a, w0_ref[...],
                                preferred_element_type=jnp.float32) + b0_ref[...])
    o_ref[...] = jax.nn.softplus(
        jnp.dot(z, w1_ref[...], preferred_element_type=jnp.float32) + b1_ref[...])


def _mlp(agg, h, w0, b0, w1, b1):
    return pl.pallas_call(
        _mlp_body,
        grid=(5,),
        in_specs=[
            pl.BlockSpec((NC, 2000, H), lambda i: (0, i, 0)),
            pl.BlockSpec((2000, H), lambda i: (i, 0)),
            pl.BlockSpec((H, H), lambda i: (0, 0)),
            pl.BlockSpec((1, H), lambda i: (0, 0)),
            pl.BlockSpec((H, H), lambda i: (0, 0)),
            pl.BlockSpec((1, H), lambda i: (0, 0)),
        ],
        out_specs=pl.BlockSpec((2000, H), lambda i: (i, 0)),
        out_shape=jax.ShapeDtypeStruct((N, H), jnp.float32),
    )(agg, h, w0, b0, w1, b1)


def _mlp_pool_body(agg_ref, h_ref, w0_ref, b0_ref, w1_ref, b1_ref, m_ref,
                   part_ref, msum_ref):
    a = agg_ref[0] + agg_ref[1] + h_ref[...]
    z = jax.nn.softplus(jnp.dot(a, w0_ref[...],
                                preferred_element_type=jnp.float32) + b0_ref[...])
    z = jnp.dot(z, w1_ref[...], preferred_element_type=jnp.float32) + b1_ref[...]
    m = m_ref[...]
    part_ref[0] = jnp.sum(z * m, axis=0, keepdims=True)
    msum_ref[0] = jnp.sum(m, axis=0, keepdims=True)


def _mlp_pool(agg, h, w0, b0, w1, b1, mask_b):
    return pl.pallas_call(
        _mlp_pool_body,
        grid=(5,),
        in_specs=[
            pl.BlockSpec((NC, 2000, H), lambda i: (0, i, 0)),
            pl.BlockSpec((2000, H), lambda i: (i, 0)),
            pl.BlockSpec((H, H), lambda i: (0, 0)),
            pl.BlockSpec((1, H), lambda i: (0, 0)),
            pl.BlockSpec((H, H), lambda i: (0, 0)),
            pl.BlockSpec((1, H), lambda i: (0, 0)),
            pl.BlockSpec((2000, H), lambda i: (i, 0)),
        ],
        out_specs=[
            pl.BlockSpec((1, 1, H), lambda i: (i, 0, 0)),
            pl.BlockSpec((1, 1, H), lambda i: (i, 0, 0)),
        ],
        out_shape=[
            jax.ShapeDtypeStruct((5, 1, H), jnp.float32),
            jax.ShapeDtypeStruct((5, 1, H), jnp.float32),
        ],
    )(agg, h, w0, b0, w1, b1, mask_b)


def _proj_body(part_ref, msum_ref, w0_ref, b0_ref, w1_ref, b1_ref, o_ref):
    pooled = jnp.sum(part_ref[...], axis=0) / jnp.sum(msum_ref[...], axis=0)
    pr = jnp.broadcast_to(pooled, (8, H))
    y = jnp.maximum(jnp.dot(pr, w0_ref[...],
                            preferred_element_type=jnp.float32) + b0_ref[...], 0.0)
    o_ref[...] = jnp.dot(y, w1_ref[...],
                         preferred_element_type=jnp.float32) + b1_ref[...]


def _proj(part, msum, w0, b0, w1p, b1p):
    return pl.pallas_call(
        _proj_body,
        out_shape=jax.ShapeDtypeStruct((8, H), jnp.float32),
    )(part, msum, w0, b0, w1p, b1p)


def kernel(node_features, batch_mask, pos, edge_index, node_W, node_b,
           edge_W0, edge_b0, edge_W1, edge_b1,
           c0_W0, c0_b0, c0_W1, c0_b1, c1_W0, c1_b0, c1_W1, c1_b1,
           c2_W0, c2_b0, c2_W1, c2_b1, proj_W0, proj_b0, proj_W1, proj_b1):
    x = node_features.reshape(N, D)
    p = pos.reshape(N, 3)
    px = jnp.pad(p[:, 0], (0, NROW - N))
    py = jnp.pad(p[:, 1], (0, NROW - N))
    pz = jnp.pad(p[:, 2], (0, NROW - N))

    row = edge_index[0]
    col = edge_index[1]
    pad = EPAD - E
    rowp = jnp.concatenate([row, jnp.zeros((pad,), jnp.int32)]).reshape(NW, NCH, 1, CHUNK)
    colp = jnp.concatenate([col, jnp.full((pad,), N, jnp.int32)]).reshape(NW, NCH, 1, CHUNK)

    h = _embed(x, node_W, node_b.reshape(1, H))
    d2 = _d2_call(px, py, pz, rowp, colp)
    ea = _ea(d2.reshape(EPAD // 128, 128), edge_W0.reshape(1, H),
             edge_b0.reshape(1, H), edge_W1, edge_b1.reshape(1, H))
    ea4 = ea.reshape(NW, NCH, CHUNK, H)

    zeros = jnp.zeros((NROW, H), jnp.float32)
    mask_b = jnp.broadcast_to(batch_mask.reshape(N, 1), (N, H))

    convs = [(c0_W0, c0_b0, c0_W1, c0_b1),
             (c1_W0, c1_b0, c1_W1, c1_b1),
             (c2_W0, c2_b0, c2_W1, c2_b1)]

    for i, (w0, b0, w1, b1) in enumerate(convs):
        agg = _conv_call(h, rowp, colp, ea4, zeros)
        if i < 2:
            h = _mlp(agg, h, w0, b0.reshape(1, H), w1, b1.reshape(1, H))
        else:
            part, msum = _mlp_pool(agg, h, w0, b0.reshape(1, H),
                                   w1, b1.reshape(1, H), mask_b)

    w1p = jnp.pad(proj_W1, ((0, 0), (0, H - 1)))
    b1p = jnp.pad(proj_b1.reshape(1, 1), ((0, 0), (0, H - 1)))
    out = _proj(part, msum, proj_W0, proj_b0.reshape(1, H), w1p, b1p)
    return out[0:1, 0:1]


# speedup vs baseline: 1.4933x; 1.1904x over previous
import functools

import jax
import jax.numpy as jnp
from jax import lax
from jax.experimental import pallas as pl
from jax.experimental.pallas import tpu as pltpu
from jax.experimental.pallas import tpu_sc as plsc

N = 10000
D = 128
H = 128
E = 320000

NC = 2
NS = 16
NW = NC * NS
CHUNK = 64
NCH = 160
EPW = NCH * CHUNK
EPAD = NW * EPW
NROW = 10016
RPT = 632
LROWS = NROW - (NS - 1) * RPT

TABM = 256
TAB_SCALE = (TABM - 1) / 32.0
TAB_CLAMP = float(TABM - 2) + 0.999


def _softplus_table():
    xs = -16.0 + jnp.arange(TABM, dtype=jnp.float32) * (32.0 / (TABM - 1))
    return jax.nn.softplus(xs)


def _softplus_lookup(x, tabv):
    u = jnp.minimum(jnp.maximum((x + 16.0) * jnp.float32(TAB_SCALE), 0.0),
                    jnp.float32(TAB_CLAMP))
    i0 = u.astype(jnp.int32)
    fr = u - i0.astype(jnp.float32)
    v0 = plsc.load_gather(tabv, [i0])
    v1 = plsc.load_gather(tabv, [i0 + 1])
    return jnp.maximum(v0 + fr * (v1 - v0), x)


def _d2_body(px, py, pz, row_hbm, col_hbm, d2_hbm,
             pxv, pyv, pzv, rowv, colv, outv):
    c = lax.axis_index("c")
    s = lax.axis_index("s")
    wid = s * NC + c
    pltpu.sync_copy(px, pxv)
    pltpu.sync_copy(py, pyv)
    pltpu.sync_copy(pz, pzv)
    pltpu.sync_copy(row_hbm.at[wid], rowv)
    pltpu.sync_copy(col_hbm.at[wid], colv)

    @pl.loop(0, EPW // 16)
    def _(t):
        j = t // (CHUNK // 16)
        k = (t % (CHUNK // 16)) * 16
        r = rowv[j, 0, pl.ds(k, 16)]
        cc = colv[j, 0, pl.ds(k, 16)]
        dx = plsc.load_gather(pxv, [r]) - plsc.load_gather(pxv, [cc])
        dy = plsc.load_gather(pyv, [r]) - plsc.load_gather(pyv, [cc])
        dz = plsc.load_gather(pzv, [r]) - plsc.load_gather(pzv, [cc])
        outv[j, pl.ds(k, 16)] = dx * dx + dy * dy + dz * dz

    pltpu.sync_copy(outv, d2_hbm.at[wid])


_d2_call = functools.partial(
    pl.kernel,
    out_type=jax.ShapeDtypeStruct((NW, NCH, CHUNK), jnp.float32),
    compiler_params=pltpu.CompilerParams(needs_layout_passes=False),
    mesh=plsc.VectorSubcoreMesh(core_axis_name="c", subcore_axis_name="s"),
    scratch_types=[
        pltpu.VMEM((NROW,), jnp.float32),
        pltpu.VMEM((NROW,), jnp.float32),
        pltpu.VMEM((NROW,), jnp.float32),
        pltpu.VMEM((NCH, 1, CHUNK), jnp.int32),
        pltpu.VMEM((NCH, 1, CHUNK), jnp.int32),
        pltpu.VMEM((NCH, CHUNK), jnp.float32),
    ],
)(_d2_body)


def _conv_body(h_hbm, row_hbm, col_hbm, ea_hbm, zeros_hbm, tab_hbm, out_hbm,
               rowb, colb, gbuf, eabuf, mbuf, tabv, aggsh,
               isem, gsem, easem, ssem):
    c = lax.axis_index("c")
    s = lax.axis_index("s")
    wid = s * NC + c
    pltpu.sync_copy(tab_hbm, tabv)

    @pl.when(s < NS - 1)
    def _():
        pltpu.sync_copy(zeros_hbm.at[pl.ds(s * RPT, RPT)],
                        aggsh.at[pl.ds(s * RPT, RPT)])

    @pl.when(s == NS - 1)
    def _():
        pltpu.sync_copy(zeros_hbm.at[pl.ds(s * RPT, LROWS)],
                        aggsh.at[pl.ds(s * RPT, LROWS)])

    plsc.subcore_barrier()

    def issue_idx(j, q):
        pltpu.async_copy(row_hbm.at[wid, j], rowb.at[q], isem.at[q])
        pltpu.async_copy(col_hbm.at[wid, j], colb.at[q], isem.at[q])

    def wait_idx(j, q):
        pltpu.make_async_copy(row_hbm.at[wid, j], rowb.at[q], isem.at[q]).wait()
        pltpu.make_async_copy(col_hbm.at[wid, j], colb.at[q], isem.at[q]).wait()

    def issue_gea(j, q, b):
        pltpu.async_copy(h_hbm.at[rowb.at[q, 0]], gbuf.at[b], gsem.at[b])
        pltpu.async_copy(ea_hbm.at[wid, j], eabuf.at[b], easem.at[b])

    def wait_gea(j, b):
        pltpu.make_async_copy(ea_hbm.at[wid, j], gbuf.at[b], gsem.at[b]).wait()
        pltpu.make_async_copy(ea_hbm.at[wid, j], eabuf.at[b], easem.at[b]).wait()

    def compute(b):
        @pl.loop(0, CHUNK, unroll=2)
        def _(i):
            for kk in range(H // 16):
                k = kk * 16
                x = gbuf[b, i, pl.ds(k, 16)] + eabuf[b, i, pl.ds(k, 16)]
                mbuf[b, i, pl.ds(k, 16)] = _softplus_lookup(x, tabv)

    def issue_scat(q, b):
        pltpu.async_copy(mbuf.at[b], aggsh.at[colb.at[q, 0]], ssem.at[b],
                         add=True)

    def wait_scat(b):
        pltpu.make_async_copy(ea_hbm.at[wid, 0], mbuf.at[b], ssem.at[b]).wait()

    issue_idx(0, 0)
    issue_idx(1, 1)
    wait_idx(0, 0)
    issue_gea(0, 0, 0)

    @pl.loop(0, NCH, step=4)
    def _(jj):
        for u in range(4):
            j = jj + u
            b = u & 1
            b1 = 1 - b

            @pl.when(j >= 2)
            def _():
                wait_scat(b)

            @pl.when(j + 2 < NCH)
            def _():
                issue_idx(j + 2, (u + 2) & 3)

            @pl.when(j + 1 < NCH)
            def _():
                wait_idx(j + 1, (u + 1) & 3)
                issue_gea(j + 1, (u + 1) & 3, b1)

            wait_gea(j, b)
            compute(b)
            issue_scat(u, b)

    wait_scat(0)
    wait_scat(1)
    plsc.subcore_barrier()

    @pl.when(s < NS - 1)
    def _():
        pltpu.sync_copy(aggsh.at[pl.ds(s * RPT, RPT)],
                        out_hbm.at[c, pl.ds(s * RPT, RPT)])

    @pl.when(s == NS - 1)
    def _():
        pltpu.sync_copy(aggsh.at[pl.ds(s * RPT, LROWS)],
                        out_hbm.at[c, pl.ds(s * RPT, LROWS)])


_conv_call = functools.partial(
    pl.kernel,
    out_type=jax.ShapeDtypeStruct((NC, NROW, H), jnp.float32),
    compiler_params=pltpu.CompilerParams(needs_layout_passes=False),
    mesh=plsc.VectorSubcoreMesh(core_axis_name="c", subcore_axis_name="s"),
    scratch_types=[
        pltpu.VMEM((4, 1, CHUNK), jnp.int32),
        pltpu.VMEM((4, 1, CHUNK), jnp.int32),
        pltpu.VMEM((2, CHUNK, H), jnp.float32),
        pltpu.VMEM((2, CHUNK, H), jnp.float32),
        pltpu.VMEM((2, CHUNK, H), jnp.float32),
        pltpu.VMEM((TABM,), jnp.float32),
        pltpu.VMEM_SHARED((NROW, H), jnp.float32),
        pltpu.SemaphoreType.DMA((4,)),
        pltpu.SemaphoreType.DMA((2,)),
        pltpu.SemaphoreType.DMA((2,)),
        pltpu.SemaphoreType.DMA((2,)),
    ],
)(_conv_body)


def _embed_body(x_ref, w_ref, b_ref, o_ref):
    o_ref[...] = (jnp.dot(x_ref[...], w_ref[...],
                          preferred_element_type=jnp.float32) + b_ref[...])


def _embed(x, w, b):
    return pl.pallas_call(
        _embed_body,
        grid=(5,),
        in_specs=[
            pl.BlockSpec((2000, D), lambda i: (i, 0)),
            pl.BlockSpec((D, H), lambda i: (0, 0)),
            pl.BlockSpec((1, H), lambda i: (0, 0)),
        ],
        out_specs=pl.BlockSpec((2000, H), lambda i: (i, 0)),
        out_shape=jax.ShapeDtypeStruct((N, H), jnp.float32),
    )(x, w, b)


def _ea_body(d2_ref, w0_ref, b0_ref, w1_ref, b1_ref, o_ref):
    w0 = w0_ref[...]
    b0 = b0_ref[...]
    w1 = w1_ref[...]
    b1 = b1_ref[...]
    for r in range(8):
        drow = d2_ref[r:r + 1, :]
        dcol = jnp.broadcast_to(drow, (128, 128)).T
        u = jnp.maximum(dcol * w0 + b0, 0.0)
        o_ref[pl.ds(r * 128, 128), :] = (
            jnp.dot(u, w1, preferred_element_type=jnp.float32) + b1)


def _ea(d2r, w0, b0, w1, b1):
    return pl.pallas_call(
        _ea_body,
        grid=(EPAD // 1024,),
        in_specs=[
            pl.BlockSpec((8, 128), lambda i: (i, 0)),
            pl.BlockSpec((1, H), lambda i: (0, 0)),
            pl.BlockSpec((1, H), lambda i: (0, 0)),
            pl.BlockSpec((H, H), lambda i: (0, 0)),
            pl.BlockSpec((1, H), lambda i: (0, 0)),
        ],
        out_specs=pl.BlockSpec((1024, H), lambda i: (i, 0)),
        out_shape=jax.ShapeDtypeStruct((EPAD, H), jnp.float32),
    )(d2r, w0, b0, w1, b1)


def _mlp_body(agg_ref, h_ref, w0_ref, b0_ref, w1_ref, b1_ref, o_ref):
    a = agg_ref[0] + agg_ref[1] + h_ref[...]
    z = jax.nn.softplus(jnp.dot(a, w0_ref[...],
                                preferred_element_type=jnp.float32) + b0_ref[...])
    o_ref[...] = jax.nn.softplus(
        jnp.dot(z, w1_ref[...], preferred_element_type=jnp.float32) + b1_ref[...])


def _mlp(agg, h, w0, b0, w1, b1):
    return pl.pallas_call(
        _mlp_body,
        grid=(5,),
        in_specs=[
            pl.BlockSpec((NC, 2000, H), lambda i: (0, i, 0)),
            pl.BlockSpec((2000, H), lambda i: (i, 0)),
            pl.BlockSpec((H, H), lambda i: (0, 0)),
            pl.BlockSpec((1, H), lambda i: (0, 0)),
            pl.BlockSpec((H, H), lambda i: (0, 0)),
            pl.BlockSpec((1, H), lambda i: (0, 0)),
        ],
        out_specs=pl.BlockSpec((2000, H), lambda i: (i, 0)),
        out_shape=jax.ShapeDtypeStruct((N, H), jnp.float32),
    )(agg, h, w0, b0, w1, b1)


def _mlp_pool_body(agg_ref, h_ref, w0_ref, b0_ref, w1_ref, b1_ref, m_ref,
                   part_ref, msum_ref):
    a = agg_ref[0] + agg_ref[1] + h_ref[...]
    z = jax.nn.softplus(jnp.dot(a, w0_ref[...],
                                preferred_element_type=jnp.float32) + b0_ref[...])
    z = jnp.dot(z, w1_ref[...], preferred_element_type=jnp.float32) + b1_ref[...]
    m = m_ref[...]
    part_ref[0] = jnp.sum(z * m, axis=0, keepdims=True)
    msum_ref[0] = jnp.sum(m, axis=0, keepdims=True)


def _mlp_pool(agg, h, w0, b0, w1, b1, mask_b):
    return pl.pallas_call(
        _mlp_pool_body,
        grid=(5,),
        in_specs=[
            pl.BlockSpec((NC, 2000, H), lambda i: (0, i, 0)),
            pl.BlockSpec((2000, H), lambda i: (i, 0)),
            pl.BlockSpec((H, H), lambda i: (0, 0)),
            pl.BlockSpec((1, H), lambda i: (0, 0)),
            pl.BlockSpec((H, H), lambda i: (0, 0)),
            pl.BlockSpec((1, H), lambda i: (0, 0)),
            pl.BlockSpec((2000, H), lambda i: (i, 0)),
        ],
        out_specs=[
            pl.BlockSpec((1, 1, H), lambda i: (i, 0, 0)),
            pl.BlockSpec((1, 1, H), lambda i: (i, 0, 0)),
        ],
        out_shape=[
            jax.ShapeDtypeStruct((5, 1, H), jnp.float32),
            jax.ShapeDtypeStruct((5, 1, H), jnp.float32),
        ],
    )(agg, h, w0, b0, w1, b1, mask_b)


def _proj_body(part_ref, msum_ref, w0_ref, b0_ref, w1_ref, b1_ref, o_ref):
    pooled = jnp.sum(part_ref[...], axis=0) / jnp.sum(msum_ref[...], axis=0)
    pr = jnp.broadcast_to(pooled, (8, H))
    y = jnp.maximum(jnp.dot(pr, w0_ref[...],
                            preferred_element_type=jnp.float32) + b0_ref[...], 0.0)
    o_ref[...] = jnp.dot(y, w1_ref[...],
                         preferred_element_type=jnp.float32) + b1_ref[...]


def _proj(part, msum, w0, b0, w1p, b1p):
    return pl.pallas_call(
        _proj_body,
        out_shape=jax.ShapeDtypeStruct((8, H), jnp.float32),
    )(part, msum, w0, b0, w1p, b1p)


def kernel(node_features, batch_mask, pos, edge_index, node_W, node_b,
           edge_W0, edge_b0, edge_W1, edge_b1,
           c0_W0, c0_b0, c0_W1, c0_b1, c1_W0, c1_b0, c1_W1, c1_b1,
           c2_W0, c2_b0, c2_W1, c2_b1, proj_W0, proj_b0, proj_W1, proj_b1):
    x = node_features.reshape(N, D)
    p = pos.reshape(N, 3)
    px = jnp.pad(p[:, 0], (0, NROW - N))
    py = jnp.pad(p[:, 1], (0, NROW - N))
    pz = jnp.pad(p[:, 2], (0, NROW - N))

    row = edge_index[0]
    col = edge_index[1]
    pad = EPAD - E
    rowp = jnp.concatenate([row, jnp.zeros((pad,), jnp.int32)]).reshape(NW, NCH, 1, CHUNK)
    colp = jnp.concatenate([col, jnp.full((pad,), N, jnp.int32)]).reshape(NW, NCH, 1, CHUNK)

    h = _embed(x, node_W, node_b.reshape(1, H))
    d2 = _d2_call(px, py, pz, rowp, colp)
    ea = _ea(d2.reshape(EPAD // 128, 128), edge_W0.reshape(1, H),
             edge_b0.reshape(1, H), edge_W1, edge_b1.reshape(1, H))
    ea4 = ea.reshape(NW, NCH, CHUNK, H)

    zeros = jnp.zeros((NROW, H), jnp.float32)
    sp_tab = _softplus_table()
    mask_b = jnp.broadcast_to(batch_mask.reshape(N, 1), (N, H))

    convs = [(c0_W0, c0_b0, c0_W1, c0_b1),
             (c1_W0, c1_b0, c1_W1, c1_b1),
             (c2_W0, c2_b0, c2_W1, c2_b1)]

    for i, (w0, b0, w1, b1) in enumerate(convs):
        agg = _conv_call(h, rowp, colp, ea4, zeros, sp_tab)
        if i < 2:
            h = _mlp(agg, h, w0, b0.reshape(1, H), w1, b1.reshape(1, H))
        else:
            part, msum = _mlp_pool(agg, h, w0, b0.reshape(1, H),
                                   w1, b1.reshape(1, H), mask_b)

    w1p = jnp.pad(proj_W1, ((0, 0), (0, H - 1)))
    b1p = jnp.pad(proj_b1.reshape(1, 1), ((0, 0), (0, H - 1)))
    out = _proj(part, msum, proj_W0, proj_b0.reshape(1, H), w1p, b1p)
    return out[0:1, 0:1]


# speedup vs baseline: 4.0040x; 2.6813x over previous
import functools

import jax
import jax.numpy as jnp
from jax import lax
from jax.experimental import pallas as pl
from jax.experimental.pallas import tpu as pltpu
from jax.experimental.pallas import tpu_sc as plsc

N = 10000
D = 128
H = 128
E = 320000

NC = 2
NS = 16
NW = NC * NS
CHUNK = 64
NCH = 160
EPW = NCH * CHUNK
EPAD = NW * EPW
NROW = 10016
RPT = 632
LROWS = NROW - (NS - 1) * RPT

TABM = 256
TAB_SCALE = (TABM - 1) / 32.0
TAB_CLAMP = float(TABM - 2) + 0.999


def _softplus_table():
    xs = -16.0 + jnp.arange(TABM, dtype=jnp.float32) * (32.0 / (TABM - 1))
    return jax.nn.softplus(xs)


def _softplus_lookup(x, tabv):
    u = jnp.minimum(jnp.maximum((x + 16.0) * jnp.float32(TAB_SCALE), 0.0),
                    jnp.float32(TAB_CLAMP))
    i0 = u.astype(jnp.int32)
    fr = u - i0.astype(jnp.float32)
    v0 = plsc.load_gather(tabv, [i0])
    v1 = plsc.load_gather(tabv, [i0 + 1])
    return jnp.maximum(v0 + fr * (v1 - v0), x)


def _d2_body(px, py, pz, row_hbm, col_hbm, d2_hbm,
             pxv, pyv, pzv, rowv, colv, outv):
    c = lax.axis_index("c")
    s = lax.axis_index("s")
    wid = s * NC + c
    pltpu.sync_copy(px, pxv)
    pltpu.sync_copy(py, pyv)
    pltpu.sync_copy(pz, pzv)
    pltpu.sync_copy(row_hbm.at[wid], rowv)
    pltpu.sync_copy(col_hbm.at[wid], colv)

    @pl.loop(0, EPW // 16)
    def _(t):
        j = t // (CHUNK // 16)
        k = (t % (CHUNK // 16)) * 16
        r = rowv[j, 0, pl.ds(k, 16)]
        cc = colv[j, 0, pl.ds(k, 16)]
        dx = plsc.load_gather(pxv, [r]) - plsc.load_gather(pxv, [cc])
        dy = plsc.load_gather(pyv, [r]) - plsc.load_gather(pyv, [cc])
        dz = plsc.load_gather(pzv, [r]) - plsc.load_gather(pzv, [cc])
        outv[j, pl.ds(k, 16)] = dx * dx + dy * dy + dz * dz

    pltpu.sync_copy(outv, d2_hbm.at[wid])


_d2_call = functools.partial(
    pl.kernel,
    out_type=jax.ShapeDtypeStruct((NW, NCH, CHUNK), jnp.float32),
    compiler_params=pltpu.CompilerParams(needs_layout_passes=False),
    mesh=plsc.VectorSubcoreMesh(core_axis_name="c", subcore_axis_name="s"),
    scratch_types=[
        pltpu.VMEM((NROW,), jnp.float32),
        pltpu.VMEM((NROW,), jnp.float32),
        pltpu.VMEM((NROW,), jnp.float32),
        pltpu.VMEM((NCH, 1, CHUNK), jnp.int32),
        pltpu.VMEM((NCH, 1, CHUNK), jnp.int32),
        pltpu.VMEM((NCH, CHUNK), jnp.float32),
    ],
)(_d2_body)


def _conv_body(h_hbm, row_hbm, col_hbm, ea_hbm, zeros_hbm, tab_hbm, out_hbm,
               rowb, colb, gbuf, eabuf, mbuf, tabv, aggsh,
               isem, gsem, easem, ssem):
    c = lax.axis_index("c")
    s = lax.axis_index("s")
    wid = s * NC + c
    pltpu.sync_copy(tab_hbm, tabv)

    @pl.when(s < NS - 1)
    def _():
        pltpu.sync_copy(zeros_hbm.at[pl.ds(s * RPT, RPT)],
                        aggsh.at[pl.ds(s * RPT, RPT)])

    @pl.when(s == NS - 1)
    def _():
        pltpu.sync_copy(zeros_hbm.at[pl.ds(s * RPT, LROWS)],
                        aggsh.at[pl.ds(s * RPT, LROWS)])

    plsc.subcore_barrier()

    def issue_idx(j, q):
        pltpu.async_copy(row_hbm.at[wid, j], rowb.at[q], isem.at[q])
        pltpu.async_copy(col_hbm.at[wid, j], colb.at[q], isem.at[q])

    def wait_idx(j, q):
        pltpu.make_async_copy(row_hbm.at[wid, j], rowb.at[q], isem.at[q]).wait()
        pltpu.make_async_copy(col_hbm.at[wid, j], colb.at[q], isem.at[q]).wait()

    def issue_gea(j, q, b):
        pltpu.async_copy(h_hbm.at[rowb.at[q, 0]], gbuf.at[b], gsem.at[b])
        pltpu.async_copy(ea_hbm.at[wid, j], eabuf.at[b], easem.at[b])

    def wait_gea(j, b):
        pltpu.make_async_copy(ea_hbm.at[wid, j], gbuf.at[b], gsem.at[b]).wait()
        pltpu.make_async_copy(ea_hbm.at[wid, j], eabuf.at[b], easem.at[b]).wait()

    def compute(b):
        @plsc.parallel_loop(0, CHUNK, unroll=4)
        def _(i):
            for kk in range(H // 16):
                k = kk * 16
                x = gbuf[b, i, pl.ds(k, 16)] + eabuf[b, i, pl.ds(k, 16)]
                mbuf[b, i, pl.ds(k, 16)] = _softplus_lookup(x, tabv)

    def issue_scat(q, b):
        pltpu.async_copy(mbuf.at[b], aggsh.at[colb.at[q, 0]], ssem.at[b],
                         add=True)

    def wait_scat(b):
        pltpu.make_async_copy(ea_hbm.at[wid, 0], mbuf.at[b], ssem.at[b]).wait()

    issue_idx(0, 0)
    issue_idx(1, 1)
    wait_idx(0, 0)
    issue_gea(0, 0, 0)

    @pl.loop(0, NCH, step=4)
    def _(jj):
        for u in range(4):
            j = jj + u
            b = u & 1
            b1 = 1 - b

            @pl.when(j >= 2)
            def _():
                wait_scat(b)

            @pl.when(j + 2 < NCH)
            def _():
                issue_idx(j + 2, (u + 2) & 3)

            @pl.when(j + 1 < NCH)
            def _():
                wait_idx(j + 1, (u + 1) & 3)
                issue_gea(j + 1, (u + 1) & 3, b1)

            wait_gea(j, b)
            compute(b)
            issue_scat(u, b)

    wait_scat(0)
    wait_scat(1)
    plsc.subcore_barrier()

    @pl.when(s < NS - 1)
    def _():
        pltpu.sync_copy(aggsh.at[pl.ds(s * RPT, RPT)],
                        out_hbm.at[c, pl.ds(s * RPT, RPT)])

    @pl.when(s == NS - 1)
    def _():
        pltpu.sync_copy(aggsh.at[pl.ds(s * RPT, LROWS)],
                        out_hbm.at[c, pl.ds(s * RPT, LROWS)])


_conv_call = functools.partial(
    pl.kernel,
    out_type=jax.ShapeDtypeStruct((NC, NROW, H), jnp.float32),
    compiler_params=pltpu.CompilerParams(needs_layout_passes=False),
    mesh=plsc.VectorSubcoreMesh(core_axis_name="c", subcore_axis_name="s"),
    scratch_types=[
        pltpu.VMEM((4, 1, CHUNK), jnp.int32),
        pltpu.VMEM((4, 1, CHUNK), jnp.int32),
        pltpu.VMEM((2, CHUNK, H), jnp.float32),
        pltpu.VMEM((2, CHUNK, H), jnp.float32),
        pltpu.VMEM((2, CHUNK, H), jnp.float32),
        pltpu.VMEM((TABM,), jnp.float32),
        pltpu.VMEM_SHARED((NROW, H), jnp.float32),
        pltpu.SemaphoreType.DMA((4,)),
        pltpu.SemaphoreType.DMA((2,)),
        pltpu.SemaphoreType.DMA((2,)),
        pltpu.SemaphoreType.DMA((2,)),
    ],
)(_conv_body)


def _embed_body(x_ref, w_ref, b_ref, o_ref):
    o_ref[...] = (jnp.dot(x_ref[...], w_ref[...],
                          preferred_element_type=jnp.float32) + b_ref[...])


def _embed(x, w, b):
    return pl.pallas_call(
        _embed_body,
        grid=(5,),
        in_specs=[
            pl.BlockSpec((2000, D), lambda i: (i, 0)),
            pl.BlockSpec((D, H), lambda i: (0, 0)),
            pl.BlockSpec((1, H), lambda i: (0, 0)),
        ],
        out_specs=pl.BlockSpec((2000, H), lambda i: (i, 0)),
        out_shape=jax.ShapeDtypeStruct((N, H), jnp.float32),
    )(x, w, b)


def _ea_body(d2_ref, w0_ref, b0_ref, w1_ref, b1_ref, o_ref):
    w0 = w0_ref[...]
    b0 = b0_ref[...]
    w1 = w1_ref[...]
    b1 = b1_ref[...]
    for r in range(8):
        drow = d2_ref[r:r + 1, :]
        dcol = jnp.broadcast_to(drow, (128, 128)).T
        u = jnp.maximum(dcol * w0 + b0, 0.0)
        o_ref[pl.ds(r * 128, 128), :] = (
            jnp.dot(u, w1, preferred_element_type=jnp.float32) + b1)


def _ea(d2r, w0, b0, w1, b1):
    return pl.pallas_call(
        _ea_body,
        grid=(EPAD // 1024,),
        in_specs=[
            pl.BlockSpec((8, 128), lambda i: (i, 0)),
            pl.BlockSpec((1, H), lambda i: (0, 0)),
            pl.BlockSpec((1, H), lambda i: (0, 0)),
            pl.BlockSpec((H, H), lambda i: (0, 0)),
            pl.BlockSpec((1, H), lambda i: (0, 0)),
        ],
        out_specs=pl.BlockSpec((1024, H), lambda i: (i, 0)),
        out_shape=jax.ShapeDtypeStruct((EPAD, H), jnp.float32),
    )(d2r, w0, b0, w1, b1)


def _mlp_body(agg_ref, h_ref, w0_ref, b0_ref, w1_ref, b1_ref, o_ref):
    a = agg_ref[0] + agg_ref[1] + h_ref[...]
    z = jax.nn.softplus(jnp.dot(a, w0_ref[...],
                                preferred_element_type=jnp.float32) + b0_ref[...])
    o_ref[...] = jax.nn.softplus(
        jnp.dot(z, w1_ref[...], preferred_element_type=jnp.float32) + b1_ref[...])


def _mlp(agg, h, w0, b0, w1, b1):
    return pl.pallas_call(
        _mlp_body,
        grid=(5,),
        in_specs=[
            pl.BlockSpec((NC, 2000, H), lambda i: (0, i, 0)),
            pl.BlockSpec((2000, H), lambda i: (i, 0)),
            pl.BlockSpec((H, H), lambda i: (0, 0)),
            pl.BlockSpec((1, H), lambda i: (0, 0)),
            pl.BlockSpec((H, H), lambda i: (0, 0)),
            pl.BlockSpec((1, H), lambda i: (0, 0)),
        ],
        out_specs=pl.BlockSpec((2000, H), lambda i: (i, 0)),
        out_shape=jax.ShapeDtypeStruct((N, H), jnp.float32),
    )(agg, h, w0, b0, w1, b1)


def _mlp_pool_body(agg_ref, h_ref, w0_ref, b0_ref, w1_ref, b1_ref, m_ref,
                   part_ref, msum_ref):
    a = agg_ref[0] + agg_ref[1] + h_ref[...]
    z = jax.nn.softplus(jnp.dot(a, w0_ref[...],
                                preferred_element_type=jnp.float32) + b0_ref[...])
    z = jnp.dot(z, w1_ref[...], preferred_element_type=jnp.float32) + b1_ref[...]
    m = m_ref[...]
    part_ref[0] = jnp.sum(z * m, axis=0, keepdims=True)
    msum_ref[0] = jnp.sum(m, axis=0, keepdims=True)


def _mlp_pool(agg, h, w0, b0, w1, b1, mask_b):
    return pl.pallas_call(
        _mlp_pool_body,
        grid=(5,),
        in_specs=[
            pl.BlockSpec((NC, 2000, H), lambda i: (0, i, 0)),
            pl.BlockSpec((2000, H), lambda i: (i, 0)),
            pl.BlockSpec((H, H), lambda i: (0, 0)),
            pl.BlockSpec((1, H), lambda i: (0, 0)),
            pl.BlockSpec((H, H), lambda i: (0, 0)),
            pl.BlockSpec((1, H), lambda i: (0, 0)),
            pl.BlockSpec((2000, H), lambda i: (i, 0)),
        ],
        out_specs=[
            pl.BlockSpec((1, 1, H), lambda i: (i, 0, 0)),
            pl.BlockSpec((1, 1, H), lambda i: (i, 0, 0)),
        ],
        out_shape=[
            jax.ShapeDtypeStruct((5, 1, H), jnp.float32),
            jax.ShapeDtypeStruct((5, 1, H), jnp.float32),
        ],
    )(agg, h, w0, b0, w1, b1, mask_b)


def _proj_body(part_ref, msum_ref, w0_ref, b0_ref, w1_ref, b1_ref, o_ref):
    pooled = jnp.sum(part_ref[...], axis=0) / jnp.sum(msum_ref[...], axis=0)
    pr = jnp.broadcast_to(pooled, (8, H))
    y = jnp.maximum(jnp.dot(pr, w0_ref[...],
                            preferred_element_type=jnp.float32) + b0_ref[...], 0.0)
    o_ref[...] = jnp.dot(y, w1_ref[...],
                         preferred_element_type=jnp.float32) + b1_ref[...]


def _proj(part, msum, w0, b0, w1p, b1p):
    return pl.pallas_call(
        _proj_body,
        out_shape=jax.ShapeDtypeStruct((8, H), jnp.float32),
    )(part, msum, w0, b0, w1p, b1p)


def kernel(node_features, batch_mask, pos, edge_index, node_W, node_b,
           edge_W0, edge_b0, edge_W1, edge_b1,
           c0_W0, c0_b0, c0_W1, c0_b1, c1_W0, c1_b0, c1_W1, c1_b1,
           c2_W0, c2_b0, c2_W1, c2_b1, proj_W0, proj_b0, proj_W1, proj_b1):
    x = node_features.reshape(N, D)
    p = pos.reshape(N, 3)
    px = jnp.pad(p[:, 0], (0, NROW - N))
    py = jnp.pad(p[:, 1], (0, NROW - N))
    pz = jnp.pad(p[:, 2], (0, NROW - N))

    row = edge_index[0]
    col = edge_index[1]
    pad = EPAD - E
    rowp = jnp.concatenate([row, jnp.zeros((pad,), jnp.int32)]).reshape(NW, NCH, 1, CHUNK)
    colp = jnp.concatenate([col, jnp.full((pad,), N, jnp.int32)]).reshape(NW, NCH, 1, CHUNK)

    h = _embed(x, node_W, node_b.reshape(1, H))
    d2 = _d2_call(px, py, pz, rowp, colp)
    ea = _ea(d2.reshape(EPAD // 128, 128), edge_W0.reshape(1, H),
             edge_b0.reshape(1, H), edge_W1, edge_b1.reshape(1, H))
    ea4 = ea.reshape(NW, NCH, CHUNK, H)

    zeros = jnp.zeros((NROW, H), jnp.float32)
    sp_tab = _softplus_table()
    mask_b = jnp.broadcast_to(batch_mask.reshape(N, 1), (N, H))

    convs = [(c0_W0, c0_b0, c0_W1, c0_b1),
             (c1_W0, c1_b0, c1_W1, c1_b1),
             (c2_W0, c2_b0, c2_W1, c2_b1)]

    for i, (w0, b0, w1, b1) in enumerate(convs):
        agg = _conv_call(h, rowp, colp, ea4, zeros, sp_tab)
        if i < 2:
            h = _mlp(agg, h, w0, b0.reshape(1, H), w1, b1.reshape(1, H))
        else:
            part, msum = _mlp_pool(agg, h, w0, b0.reshape(1, H),
                                   w1, b1.reshape(1, H), mask_b)

    w1p = jnp.pad(proj_W1, ((0, 0), (0, H - 1)))
    b1p = jnp.pad(proj_b1.reshape(1, 1), ((0, 0), (0, H - 1)))
    out = _proj(part, msum, proj_W0, proj_b0.reshape(1, H), w1p, b1p)
    return out[0:1, 0:1]


# speedup vs baseline: 4.0271x; 1.0058x over previous
import functools

import jax
import jax.numpy as jnp
from jax import lax
from jax.experimental import pallas as pl
from jax.experimental.pallas import tpu as pltpu
from jax.experimental.pallas import tpu_sc as plsc

N = 10000
D = 128
H = 128
E = 320000

NC = 2
NS = 16
NW = NC * NS
CHUNK = 64
NCH = 160
EPW = NCH * CHUNK
EPAD = NW * EPW
NROW = 10016
RPT = 632
LROWS = NROW - (NS - 1) * RPT

TABM = 512
TAB_SCALE = (TABM - 1) / 32.0
TAB_CLAMP = float(TABM - 2) + 0.999


def _softplus_table():
    xs = -16.0 + jnp.arange(TABM, dtype=jnp.float32) * (32.0 / (TABM - 1))
    return jax.nn.softplus(xs)


def _softplus_lookup(x, tabv):
    u = jnp.minimum(jnp.maximum((x + 16.0) * jnp.float32(TAB_SCALE), 0.0),
                    jnp.float32(TAB_CLAMP))
    i0 = u.astype(jnp.int32)
    fr = u - i0.astype(jnp.float32)
    v0 = plsc.load_gather(tabv, [i0])
    v1 = plsc.load_gather(tabv, [i0 + 1])
    return jnp.maximum(v0 + fr * (v1 - v0), x)


def _d2_body(px, py, pz, row_hbm, col_hbm, d2_hbm,
             pxv, pyv, pzv, rowv, colv, outv):
    c = lax.axis_index("c")
    s = lax.axis_index("s")
    wid = s * NC + c
    pltpu.sync_copy(px, pxv)
    pltpu.sync_copy(py, pyv)
    pltpu.sync_copy(pz, pzv)
    pltpu.sync_copy(row_hbm.at[wid], rowv)
    pltpu.sync_copy(col_hbm.at[wid], colv)

    @pl.loop(0, EPW // 16)
    def _(t):
        j = t // (CHUNK // 16)
        k = (t % (CHUNK // 16)) * 16
        r = rowv[j, 0, pl.ds(k, 16)]
        cc = colv[j, 0, pl.ds(k, 16)]
        dx = plsc.load_gather(pxv, [r]) - plsc.load_gather(pxv, [cc])
        dy = plsc.load_gather(pyv, [r]) - plsc.load_gather(pyv, [cc])
        dz = plsc.load_gather(pzv, [r]) - plsc.load_gather(pzv, [cc])
        outv[j, pl.ds(k, 16)] = dx * dx + dy * dy + dz * dz

    pltpu.sync_copy(outv, d2_hbm.at[wid])


_d2_call = functools.partial(
    pl.kernel,
    out_type=jax.ShapeDtypeStruct((NW, NCH, CHUNK), jnp.float32),
    compiler_params=pltpu.CompilerParams(needs_layout_passes=False),
    mesh=plsc.VectorSubcoreMesh(core_axis_name="c", subcore_axis_name="s"),
    scratch_types=[
        pltpu.VMEM((NROW,), jnp.float32),
        pltpu.VMEM((NROW,), jnp.float32),
        pltpu.VMEM((NROW,), jnp.float32),
        pltpu.VMEM((NCH, 1, CHUNK), jnp.int32),
        pltpu.VMEM((NCH, 1, CHUNK), jnp.int32),
        pltpu.VMEM((NCH, CHUNK), jnp.float32),
    ],
)(_d2_body)


def _conv_body(h_hbm, row_hbm, col_hbm, ea_hbm, zeros_hbm, tab_hbm, out_hbm,
               rowb, colb, gbuf, eabuf, mbuf, tabv, aggsh,
               isem, gsem, ssem):
    c = lax.axis_index("c")
    s = lax.axis_index("s")
    wid = s * NC + c
    pltpu.sync_copy(tab_hbm, tabv)

    @pl.when(s < NS - 1)
    def _():
        pltpu.sync_copy(zeros_hbm.at[pl.ds(s * RPT, RPT)],
                        aggsh.at[pl.ds(s * RPT, RPT)])

    @pl.when(s == NS - 1)
    def _():
        pltpu.sync_copy(zeros_hbm.at[pl.ds(s * RPT, LROWS)],
                        aggsh.at[pl.ds(s * RPT, LROWS)])

    plsc.subcore_barrier()

    def issue_idx(j, q):
        pltpu.async_copy(row_hbm.at[wid, j], rowb.at[q], isem.at[q])
        pltpu.async_copy(col_hbm.at[wid, j], colb.at[q], isem.at[q])

    def wait_idx(j, q):
        pltpu.make_async_copy(row_hbm.at[wid, j], rowb.at[q], isem.at[q]).wait()
        pltpu.make_async_copy(col_hbm.at[wid, j], colb.at[q], isem.at[q]).wait()

    def issue_gea(j, q, b):
        pltpu.async_copy(h_hbm.at[rowb.at[q, 0]], gbuf.at[b], gsem.at[b])
        pltpu.async_copy(ea_hbm.at[wid, j], eabuf.at[b], gsem.at[b])

    def wait_gea(j, b):
        pltpu.make_async_copy(ea_hbm.at[wid, pl.ds(0, 2)], gbuf,
                              gsem.at[b]).wait()

    def compute(b):
        @plsc.parallel_loop(0, CHUNK, unroll=4)
        def _(i):
            for kk in range(H // 16):
                k = kk * 16
                x = gbuf[b, i, pl.ds(k, 16)] + eabuf[b, i, pl.ds(k, 16)]
                mbuf[b, i, pl.ds(k, 16)] = _softplus_lookup(x, tabv)

    def issue_scat(q, b):
        pltpu.async_copy(mbuf.at[b], aggsh.at[colb.at[q, 0]], ssem.at[b],
                         add=True)

    def wait_scat(b):
        pltpu.make_async_copy(ea_hbm.at[wid, 0], mbuf.at[b], ssem.at[b]).wait()

    issue_idx(0, 0)
    issue_idx(1, 1)
    wait_idx(0, 0)
    issue_gea(0, 0, 0)

    @pl.loop(0, NCH, step=4)
    def _(jj):
        for u in range(4):
            j = jj + u
            b = u & 1
            b1 = 1 - b

            @pl.when(j >= 2)
            def _():
                wait_scat(b)

            @pl.when(j + 2 < NCH)
            def _():
                issue_idx(j + 2, (u + 2) & 3)

            @pl.when(j + 1 < NCH)
            def _():
                wait_idx(j + 1, (u + 1) & 3)
                issue_gea(j + 1, (u + 1) & 3, b1)

            wait_gea(j, b)
            compute(b)
            issue_scat(u, b)

    wait_scat(0)
    wait_scat(1)
    plsc.subcore_barrier()

    @pl.when(s < NS - 1)
    def _():
        pltpu.sync_copy(aggsh.at[pl.ds(s * RPT, RPT)],
                        out_hbm.at[c, pl.ds(s * RPT, RPT)])

    @pl.when(s == NS - 1)
    def _():
        pltpu.sync_copy(aggsh.at[pl.ds(s * RPT, LROWS)],
                        out_hbm.at[c, pl.ds(s * RPT, LROWS)])


_conv_call = functools.partial(
    pl.kernel,
    out_type=jax.ShapeDtypeStruct((NC, NROW, H), jnp.float32),
    compiler_params=pltpu.CompilerParams(needs_layout_passes=False),
    mesh=plsc.VectorSubcoreMesh(core_axis_name="c", subcore_axis_name="s"),
    scratch_types=[
        pltpu.VMEM((4, 1, CHUNK), jnp.int32),
        pltpu.VMEM((4, 1, CHUNK), jnp.int32),
        pltpu.VMEM((2, CHUNK, H), jnp.float32),
        pltpu.VMEM((2, CHUNK, H), jnp.float32),
        pltpu.VMEM((2, CHUNK, H), jnp.float32),
        pltpu.VMEM((TABM,), jnp.float32),
        pltpu.VMEM_SHARED((NROW, H), jnp.float32),
        pltpu.SemaphoreType.DMA((4,)),
        pltpu.SemaphoreType.DMA((2,)),
        pltpu.SemaphoreType.DMA((2,)),
    ],
)(_conv_body)


def _embed_body(x_ref, w_ref, b_ref, o_ref):
    o_ref[...] = (jnp.dot(x_ref[...], w_ref[...],
                          preferred_element_type=jnp.float32) + b_ref[...])


def _embed(x, w, b):
    return pl.pallas_call(
        _embed_body,
        grid=(5,),
        in_specs=[
            pl.BlockSpec((2000, D), lambda i: (i, 0)),
            pl.BlockSpec((D, H), lambda i: (0, 0)),
            pl.BlockSpec((1, H), lambda i: (0, 0)),
        ],
        out_specs=pl.BlockSpec((2000, H), lambda i: (i, 0)),
        out_shape=jax.ShapeDtypeStruct((N, H), jnp.float32),
    )(x, w, b)


def _ea_body(d2_ref, w0_ref, b0_ref, w1_ref, b1_ref, o_ref):
    w0 = w0_ref[...]
    b0 = b0_ref[...]
    w1 = w1_ref[...]
    b1 = b1_ref[...]
    for r in range(8):
        drow = d2_ref[r:r + 1, :]
        dcol = jnp.broadcast_to(drow, (128, 128)).T
        u = jnp.maximum(dcol * w0 + b0, 0.0)
        o_ref[pl.ds(r * 128, 128), :] = (
            jnp.dot(u, w1, preferred_element_type=jnp.float32) + b1)


def _ea(d2r, w0, b0, w1, b1):
    return pl.pallas_call(
        _ea_body,
        grid=(EPAD // 1024,),
        in_specs=[
            pl.BlockSpec((8, 128), lambda i: (i, 0)),
            pl.BlockSpec((1, H), lambda i: (0, 0)),
            pl.BlockSpec((1, H), lambda i: (0, 0)),
            pl.BlockSpec((H, H), lambda i: (0, 0)),
            pl.BlockSpec((1, H), lambda i: (0, 0)),
        ],
        out_specs=pl.BlockSpec((1024, H), lambda i: (i, 0)),
        out_shape=jax.ShapeDtypeStruct((EPAD, H), jnp.float32),
    )(d2r, w0, b0, w1, b1)


def _mlp_body(agg_ref, h_ref, w0_ref, b0_ref, w1_ref, b1_ref, o_ref):
    a = agg_ref[0] + agg_ref[1] + h_ref[...]
    z = jax.nn.softplus(jnp.dot(a, w0_ref[...],
                                preferred_element_type=jnp.float32) + b0_ref[...])
    o_ref[...] = jax.nn.softplus(
        jnp.dot(z, w1_ref[...], preferred_element_type=jnp.float32) + b1_ref[...])


def _mlp(agg, h, w0, b0, w1, b1):
    return pl.pallas_call(
        _mlp_body,
        grid=(5,),
        in_specs=[
            pl.BlockSpec((NC, 2000, H), lambda i: (0, i, 0)),
            pl.BlockSpec((2000, H), lambda i: (i, 0)),
            pl.BlockSpec((H, H), lambda i: (0, 0)),
            pl.BlockSpec((1, H), lambda i: (0, 0)),
            pl.BlockSpec((H, H), lambda i: (0, 0)),
            pl.BlockSpec((1, H), lambda i: (0, 0)),
        ],
        out_specs=pl.BlockSpec((2000, H), lambda i: (i, 0)),
        out_shape=jax.ShapeDtypeStruct((N, H), jnp.float32),
    )(agg, h, w0, b0, w1, b1)


def _mlp_pool_body(agg_ref, h_ref, w0_ref, b0_ref, w1_ref, b1_ref, m_ref,
                   part_ref, msum_ref):
    a = agg_ref[0] + agg_ref[1] + h_ref[...]
    z = jax.nn.softplus(jnp.dot(a, w0_ref[...],
                                preferred_element_type=jnp.float32) + b0_ref[...])
    z = jnp.dot(z, w1_ref[...], preferred_element_type=jnp.float32) + b1_ref[...]
    m = m_ref[...]
    part_ref[0] = jnp.sum(z * m, axis=0, keepdims=True)
    msum_ref[0] = jnp.sum(m, axis=0, keepdims=True)


def _mlp_pool(agg, h, w0, b0, w1, b1, mask_b):
    return pl.pallas_call(
        _mlp_pool_body,
        grid=(5,),
        in_specs=[
            pl.BlockSpec((NC, 2000, H), lambda i: (0, i, 0)),
            pl.BlockSpec((2000, H), lambda i: (i, 0)),
            pl.BlockSpec((H, H), lambda i: (0, 0)),
            pl.BlockSpec((1, H), lambda i: (0, 0)),
            pl.BlockSpec((H, H), lambda i: (0, 0)),
            pl.BlockSpec((1, H), lambda i: (0, 0)),
            pl.BlockSpec((2000, H), lambda i: (i, 0)),
        ],
        out_specs=[
            pl.BlockSpec((1, 1, H), lambda i: (i, 0, 0)),
            pl.BlockSpec((1, 1, H), lambda i: (i, 0, 0)),
        ],
        out_shape=[
            jax.ShapeDtypeStruct((5, 1, H), jnp.float32),
            jax.ShapeDtypeStruct((5, 1, H), jnp.float32),
        ],
    )(agg, h, w0, b0, w1, b1, mask_b)


def _proj_body(part_ref, msum_ref, w0_ref, b0_ref, w1_ref, b1_ref, o_ref):
    pooled = jnp.sum(part_ref[...], axis=0) / jnp.sum(msum_ref[...], axis=0)
    pr = jnp.broadcast_to(pooled, (8, H))
    y = jnp.maximum(jnp.dot(pr, w0_ref[...],
                            preferred_element_type=jnp.float32) + b0_ref[...], 0.0)
    o_ref[...] = jnp.dot(y, w1_ref[...],
                         preferred_element_type=jnp.float32) + b1_ref[...]


def _proj(part, msum, w0, b0, w1p, b1p):
    return pl.pallas_call(
        _proj_body,
        out_shape=jax.ShapeDtypeStruct((8, H), jnp.float32),
    )(part, msum, w0, b0, w1p, b1p)


def kernel(node_features, batch_mask, pos, edge_index, node_W, node_b,
           edge_W0, edge_b0, edge_W1, edge_b1,
           c0_W0, c0_b0, c0_W1, c0_b1, c1_W0, c1_b0, c1_W1, c1_b1,
           c2_W0, c2_b0, c2_W1, c2_b1, proj_W0, proj_b0, proj_W1, proj_b1):
    x = node_features.reshape(N, D)
    p = pos.reshape(N, 3)
    px = jnp.pad(p[:, 0], (0, NROW - N))
    py = jnp.pad(p[:, 1], (0, NROW - N))
    pz = jnp.pad(p[:, 2], (0, NROW - N))

    row = edge_index[0]
    col = edge_index[1]
    pad = EPAD - E
    rowp = jnp.concatenate([row, jnp.zeros((pad,), jnp.int32)]).reshape(NW, NCH, 1, CHUNK)
    colp = jnp.concatenate([col, jnp.full((pad,), N, jnp.int32)]).reshape(NW, NCH, 1, CHUNK)

    h = _embed(x, node_W, node_b.reshape(1, H))
    d2 = _d2_call(px, py, pz, rowp, colp)
    ea = _ea(d2.reshape(EPAD // 128, 128), edge_W0.reshape(1, H),
             edge_b0.reshape(1, H), edge_W1, edge_b1.reshape(1, H))
    ea4 = ea.reshape(NW, NCH, CHUNK, H)

    zeros = jnp.zeros((NROW, H), jnp.float32)
    sp_tab = _softplus_table()
    mask_b = jnp.broadcast_to(batch_mask.reshape(N, 1), (N, H))

    convs = [(c0_W0, c0_b0, c0_W1, c0_b1),
             (c1_W0, c1_b0, c1_W1, c1_b1),
             (c2_W0, c2_b0, c2_W1, c2_b1)]

    for i, (w0, b0, w1, b1) in enumerate(convs):
        agg = _conv_call(h, rowp, colp, ea4, zeros, sp_tab)
        if i < 2:
            h = _mlp(agg, h, w0, b0.reshape(1, H), w1, b1.reshape(1, H))
        else:
            part, msum = _mlp_pool(agg, h, w0, b0.reshape(1, H),
                                   w1, b1.reshape(1, H), mask_b)

    w1p = jnp.pad(proj_W1, ((0, 0), (0, H - 1)))
    b1p = jnp.pad(proj_b1.reshape(1, 1), ((0, 0), (0, H - 1)))
    out = _proj(part, msum, proj_W0, proj_b0.reshape(1, H), w1p, b1p)
    return out[0:1, 0:1]
